# Initial kernel scaffold; baseline (speedup 1.0000x reference)
import jax
import jax.numpy as jnp
from jax.experimental import pallas as pl


def kernel(query_entities, query_timestamps, query_relations, adj_rel, adj_ent, adj_ts, ent_table, rel_table, time_w, time_b, W_x, W_h, b_g, W_score, b_score):
    raise NotImplementedError("write your pallas kernel here")


# speedup vs baseline: 1.2357x; 1.2357x over previous
import functools

import jax
import jax.numpy as jnp
from jax import lax
from jax.experimental import pallas as pl
from jax.experimental.pallas import tpu as pltpu
from jax.experimental.pallas import tpu_sc as plsc

NUM_ENT = 100000
NUM_REL = 460
ENT_DIM = 64
STATE_DIM = 128
PATH_LEN = 3
A = 50
B = 4096
NW = 32
BPW = B // NW
L = 16
AC = 256


def _gather_body(cur_e_hbm, adjcat_hbm, entp_hbm,
                 adj_o, ent_o,
                 ce_v, adjbuf, entbuf, sems, adjsem):
    wid = lax.axis_index("s") * 2 + lax.axis_index("c")
    base = wid * BPW
    iota = lax.iota(jnp.int32, L)
    lane_info = []
    for c in range(4):
        idx = iota + (L * c)
        lane_info.append(jnp.where(idx < A, idx, A - 1))

    pltpu.sync_copy(cur_e_hbm.at[pl.ds(base, BPW)], ce_v)

    def _adj_copy(g):
        eidx = plsc.load_gather(ce_v, [iota + g * L])
        eidx = jnp.clip(eidx, 0, NUM_ENT - 1)
        return pltpu.make_async_copy(adjcat_hbm.at[eidx],
                                     adjbuf.at[pl.ds(g * L, L)], adjsem)

    for g in range(BPW // L):
        _adj_copy(g).start()
    for g in range(BPW // L):
        _adj_copy(g).wait()
    pltpu.sync_copy(adjbuf, adj_o.at[pl.ds(base, BPW)])

    def _copies(b, phase):
        sem = sems.at[phase]
        bb = jnp.full((L,), b, jnp.int32)
        cps = []
        for c in range(4):
            idxe = plsc.load_gather(adjbuf, [bb, lane_info[c] + A])
            idxe = jnp.clip(idxe, 0, NUM_ENT - 1)
            cps.append(pltpu.make_async_copy(
                entp_hbm.at[idxe],
                entbuf.at[pl.ds(phase * 64 + c * L, L)], sem))
        return cps

    def _issue(b, phase):
        for cp in _copies(b, phase):
            cp.start()

    def _drain(b, phase):
        for cp in _copies(b, phase):
            cp.wait()
        pltpu.sync_copy(entbuf.at[pl.ds(phase * 64, 56)],
                        ent_o.at[pl.ds((base + b) * 56, 56)])

    _issue(0, 0)
    _issue(1, 1)

    def loop_body(i, carry):
        for j in range(2):
            b = i * 2 + j
            _drain(b, j)
            nb = b + 2

            @pl.when(nb < BPW)
            def _():
                _issue(nb, j)
        return carry

    lax.fori_loop(0, BPW // 2, loop_body, 0)


_gather = functools.partial(
    pl.kernel,
    out_type=[
        jax.ShapeDtypeStruct((B, AC), jnp.int32),
        jax.ShapeDtypeStruct((B * 56, 128), jnp.float32),
    ],
    mesh=plsc.VectorSubcoreMesh(core_axis_name="c", subcore_axis_name="s",
                                num_cores=2, num_subcores=16),
    compiler_params=pltpu.CompilerParams(needs_layout_passes=False),
    scratch_types=[
        pltpu.VMEM((BPW,), jnp.int32),
        pltpu.VMEM((BPW, AC), jnp.int32),
        pltpu.VMEM((2 * 64, 128), jnp.float32),
        pltpu.SemaphoreType.DMA((2,)),
        pltpu.SemaphoreType.DMA,
    ],
)(_gather_body)


def kernel(query_entities, query_timestamps, query_relations,
           adj_rel, adj_ent, adj_ts,
           ent_table, rel_table, time_w, time_b,
           W_x, W_h, b_g, W_score, b_score):
    f32 = jnp.float32
    qt = query_timestamps

    def ent_embs(e, dt):
        t_emb = jnp.cos(dt.astype(f32)[..., None] * time_w + time_b)
        return jnp.take(ent_table, e, axis=0) + t_emb

    def gru(x, h):
        g = x @ W_x + b_g
        gh = h @ W_h
        xr, xz, xn = jnp.split(g, 3, axis=-1)
        hr, hz, hn = jnp.split(gh, 3, axis=-1)
        r = jax.nn.sigmoid(xr + hr)
        z = jax.nn.sigmoid(xz + hz)
        n = jnp.tanh(xn + r * hn)
        return (1.0 - z) * n + z * h

    adjcat = jnp.concatenate(
        [adj_rel, adj_ent, adj_ts,
         jnp.zeros((NUM_ENT, AC - 3 * A), jnp.int32)], axis=1)
    entp = jnp.pad(ent_table, ((0, 0), (0, 128 - ENT_DIM)))

    q_ent_emb = ent_embs(query_entities, jnp.zeros_like(qt))
    q_rel_emb = jnp.take(rel_table, query_relations, axis=0)
    cur_e, cur_t = query_entities, qt
    prev_r = jnp.full_like(query_relations, NUM_REL)
    hx = jnp.zeros((B, STATE_DIM), f32)
    all_loss, all_logits, all_idx = [], [], []
    for _ in range(PATH_LEN):
        adj_rows, ent_rows = _gather(jnp.clip(cur_e, 0, NUM_ENT - 1),
                                     adjcat, entp)
        a_rel = adj_rows[:, :A]
        a_ent = adj_rows[:, A:2 * A]
        a_ts = adj_rows[:, 2 * A:3 * A]
        prev_rel_emb = jnp.take(rel_table, prev_r, axis=0)
        cur_emb = ent_embs(cur_e, qt - cur_t)
        hx = gru(jnp.concatenate([prev_rel_emb, cur_emb], axis=-1), hx)
        cand_rel = jnp.take(rel_table, a_rel, axis=0)
        t_emb = jnp.cos((qt[:, None] - a_ts).astype(f32)[..., None]
                        * time_w + time_b)
        cand_ent = (ent_rows.reshape(B, 56, 128)[:, :A, :ENT_DIM] + t_emb)
        cand = jnp.concatenate([cand_rel, cand_ent], axis=-1)
        s = jnp.concatenate([hx, q_rel_emb, q_ent_emb], axis=-1) @ W_score + b_score
        scores = jnp.einsum('bad,bd->ba', cand, s)
        valid = a_ts <= qt[:, None]
        scores = jnp.where(valid, scores, jnp.float32(-1e9))
        logits = jax.nn.log_softmax(scores, axis=1)
        action_id = jnp.argmax(logits, axis=1)[:, None]
        loss = -jnp.take_along_axis(logits, action_id, axis=1).squeeze(1)
        chosen_rel = jnp.take_along_axis(a_rel, action_id, axis=1).squeeze(1)
        chosen_ent = jnp.take_along_axis(a_ent, action_id, axis=1).squeeze(1)
        chosen_ts = jnp.take_along_axis(a_ts, action_id, axis=1).squeeze(1)
        all_loss.append(loss)
        all_logits.append(logits)
        all_idx.append(action_id.squeeze(1))
        cur_e, cur_t, prev_r = chosen_ent, chosen_ts, chosen_rel
    return (jnp.stack(all_loss), jnp.stack(all_logits), jnp.stack(all_idx),
            cur_e, cur_t)
